# Initial kernel scaffold; baseline (speedup 1.0000x reference)
import jax
import jax.numpy as jnp
from jax.experimental import pallas as pl


def kernel(key, cache_next):
    raise NotImplementedError("write your pallas kernel here")


# speedup vs baseline: 1.0222x; 1.0222x over previous
import jax
import jax.numpy as jnp
from jax.experimental import pallas as pl
from jax.experimental.pallas import tpu as pltpu


def _rmw_block(key_ref, cache_ref, out_ref):
    blk = cache_ref[...]
    rows = jax.lax.broadcasted_iota(jnp.int32, blk.shape, 1)
    cols = jax.lax.broadcasted_iota(jnp.int32, blk.shape, 2)
    upd = jnp.where((rows == 0) & (cols == 1), 2.0 * key_ref[0], 0.0)
    out_ref[...] = blk + upd


def kernel(key, cache_next):
    updated = pl.pallas_call(
        _rmw_block,
        grid=(1,),
        in_specs=[
            pl.BlockSpec(memory_space=pltpu.SMEM),
            pl.BlockSpec((1, 8, 128), lambda i: (1, 0, 0)),
        ],
        out_specs=pl.BlockSpec((1, 8, 128), lambda i: (1, 0, 0)),
        out_shape=jax.ShapeDtypeStruct(cache_next.shape, cache_next.dtype),
        input_output_aliases={1: 0},
    )(key, cache_next)
    return key, updated
